# initial kernel scaffold (unmeasured)
import jax
import jax.numpy as jnp
from jax import lax
from jax.experimental import pallas as pl
from jax.experimental.pallas import tpu as pltpu

N_DEV = 32
M = 4096
N_OUT = 2048
CH = M // N_DEV
SLOTS = 4


def kernel(x, w_mat, scale_x, scale_w):
    k_per = x.shape[1]
    assert x.shape == (M, k_per)
    assert w_mat.shape == (k_per, N_OUT)

    def body(x_ref, w_ref, sx_ref, sw_ref, out_ref,
             comm_ref, first_ref,
             send_sems, rs_recv_sems, ag_recv_sems, ready_sem):
        my = lax.axis_index("i")
        left = (my - 1) % N_DEV
        right = (my + 1) % N_DEV

        barrier = pltpu.get_barrier_semaphore()
        for nbr in (left, right):
            pl.semaphore_signal(
                barrier, inc=1,
                device_id=(nbr,), device_id_type=pl.DeviceIdType.MESH,
            )
        pl.semaphore_wait(barrier, 2)

        scale = sx_ref[0] * sw_ref[0]

        def partial_chunk(c):
            xa = x_ref[pl.ds(c * CH, CH), :]
            return lax.dot_general(
                xa, w_ref[:, :],
                dimension_numbers=(((1,), (0,)), ((), ())),
                preferred_element_type=jnp.int32,
            )

        first_ref[:, :] = partial_chunk(my)

        for s in range(N_DEV - 1):
            if s >= SLOTS:
                pl.semaphore_wait(ready_sem, 1)
            src = first_ref if s == 0 else comm_ref.at[(s - 1) % SLOTS]
            rdma = pltpu.make_async_remote_copy(
                src_ref=src,
                dst_ref=comm_ref.at[s % SLOTS],
                send_sem=send_sems.at[s % 2],
                recv_sem=rs_recv_sems.at[s],
                device_id=(right,),
                device_id_type=pl.DeviceIdType.MESH,
            )
            rdma.start()
            rdma.wait()
            c_recv = (my - 1 - s) % N_DEV
            comm_ref[s % SLOTS] = comm_ref[s % SLOTS] + partial_chunk(c_recv)
            if 1 <= s <= N_DEV - 5:
                pl.semaphore_signal(
                    ready_sem, inc=1,
                    device_id=(left,), device_id_type=pl.DeviceIdType.MESH,
                )

        g = (my + 1) % N_DEV
        acc = comm_ref[(N_DEV - 2) % SLOTS]
        y = jnp.maximum(acc.astype(jnp.float32) * scale, 0.0)
        out_ref[pl.ds(g * CH, CH), :] = y

        for s in range(N_DEV - 1):
            c_send = (my + 1 - s) % N_DEV
            rows = pl.ds(c_send * CH, CH)
            rdma = pltpu.make_async_remote_copy(
                src_ref=out_ref.at[rows, :],
                dst_ref=out_ref.at[rows, :],
                send_sem=send_sems.at[s % 2],
                recv_sem=ag_recv_sems.at[s],
                device_id=(right,),
                device_id_type=pl.DeviceIdType.MESH,
            )
            rdma.start()
            rdma.wait()

    return pl.pallas_call(
        body,
        out_shape=jax.ShapeDtypeStruct((M, N_OUT), jnp.float32),
        in_specs=[
            pl.BlockSpec(memory_space=pltpu.VMEM),
            pl.BlockSpec(memory_space=pltpu.VMEM),
            pl.BlockSpec(memory_space=pltpu.SMEM),
            pl.BlockSpec(memory_space=pltpu.SMEM),
        ],
        out_specs=pl.BlockSpec(memory_space=pltpu.VMEM),
        scratch_shapes=[
            pltpu.VMEM((SLOTS, CH, N_OUT), jnp.int32),
            pltpu.VMEM((CH, N_OUT), jnp.int32),
            pltpu.SemaphoreType.DMA((2,)),
            pltpu.SemaphoreType.DMA((N_DEV - 1,)),
            pltpu.SemaphoreType.DMA((N_DEV - 1,)),
            pltpu.SemaphoreType.REGULAR,
        ],
        compiler_params=pltpu.CompilerParams(collective_id=0),
    )(x, w_mat, scale_x, scale_w)


# baseline (device time: 861094 ns/iter reference)
import jax
import jax.numpy as jnp
from jax import lax
from jax.experimental import pallas as pl
from jax.experimental.pallas import tpu as pltpu

N_DEV = 32
M = 4096
N_OUT = 2048
CH = M // N_DEV
SLOTS = 4


def kernel(x, w_mat, scale_x, scale_w):
    k_per = x.shape[1]
    assert x.shape == (M, k_per)
    assert w_mat.shape == (k_per, N_OUT)

    def body(x_ref, w_ref, sx_ref, sw_ref, out_ref,
             comm_ref, first_ref,
             send_sems, rs_recv_sems, ag_recv_sems, ready_sem):
        my = lax.axis_index("i")
        left = (my - 1) % N_DEV
        right = (my + 1) % N_DEV

        barrier = pltpu.get_barrier_semaphore()
        for nbr in (left, right):
            pl.semaphore_signal(
                barrier, inc=1,
                device_id=(nbr,), device_id_type=pl.DeviceIdType.MESH,
            )
        pl.semaphore_wait(barrier, 2)

        scale = sx_ref[0] * sw_ref[0]

        def partial_chunk(c):
            xa = x_ref[pl.ds(c * CH, CH), :]
            return lax.dot_general(
                xa, w_ref[:, :],
                dimension_numbers=(((1,), (0,)), ((), ())),
                preferred_element_type=jnp.int32,
            )

        first_ref[:, :] = partial_chunk(my)

        for s in range(N_DEV - 1):
            if s >= SLOTS:
                pl.semaphore_wait(ready_sem, 1)
            src = first_ref if s == 0 else comm_ref.at[(s - 1) % SLOTS]
            rdma = pltpu.make_async_remote_copy(
                src_ref=src,
                dst_ref=comm_ref.at[s % SLOTS],
                send_sem=send_sems.at[s % 2],
                recv_sem=rs_recv_sems.at[s],
                device_id=(right,),
                device_id_type=pl.DeviceIdType.MESH,
            )
            rdma.start()
            rdma.wait()
            c_recv = (my - 1 - s) % N_DEV
            comm_ref[s % SLOTS] = comm_ref[s % SLOTS] + partial_chunk(c_recv)
            if 1 <= s <= N_DEV - 5:
                pl.semaphore_signal(
                    ready_sem, inc=1,
                    device_id=(left,), device_id_type=pl.DeviceIdType.MESH,
                )

        g = (my + 1) % N_DEV
        acc = comm_ref[(N_DEV - 2) % SLOTS]
        y = jnp.maximum(acc.astype(jnp.float32) * scale, 0.0)
        out_ref[pl.ds(g * CH, CH), :] = y

        for s in range(N_DEV - 1):
            c_send = (my + 1 - s) % N_DEV
            rows = pl.ds(c_send * CH, CH)
            rdma = pltpu.make_async_remote_copy(
                src_ref=out_ref.at[rows, :],
                dst_ref=out_ref.at[rows, :],
                send_sem=send_sems.at[s % 2],
                recv_sem=ag_recv_sems.at[s],
                device_id=(right,),
                device_id_type=pl.DeviceIdType.MESH,
            )
            rdma.start()
            rdma.wait()

    return pl.pallas_call(
        body,
        out_shape=jax.ShapeDtypeStruct((M, N_OUT), jnp.float32),
        in_specs=[
            pl.BlockSpec(memory_space=pltpu.VMEM),
            pl.BlockSpec(memory_space=pltpu.VMEM),
            pl.BlockSpec(memory_space=pltpu.SMEM),
            pl.BlockSpec(memory_space=pltpu.SMEM),
        ],
        out_specs=pl.BlockSpec(memory_space=pltpu.VMEM),
        scratch_shapes=[
            pltpu.VMEM((SLOTS, CH, N_OUT), jnp.int32),
            pltpu.VMEM((CH, N_OUT), jnp.int32),
            pltpu.SemaphoreType.DMA((2,)),
            pltpu.SemaphoreType.DMA((N_DEV - 1,)),
            pltpu.SemaphoreType.DMA((N_DEV - 1,)),
            pltpu.SemaphoreType.REGULAR,
        ],
        compiler_params=pltpu.CompilerParams(
            collective_id=0,
            vmem_limit_bytes=56 * 1024 * 1024,
        ),
    )(x, w_mat, scale_x, scale_w)


# device time: 844660 ns/iter; 1.0195x vs baseline; 1.0195x over previous
import jax
import jax.numpy as jnp
from jax import lax
from jax.experimental import pallas as pl
from jax.experimental.pallas import tpu as pltpu

N_DEV = 32
M = 4096
N_OUT = 2048
HALF = N_OUT // 2
CH = M // N_DEV
SLOTS = 4


def kernel(x, w_mat, scale_x, scale_w):
    k_per = x.shape[1]
    assert x.shape == (M, k_per)
    assert w_mat.shape == (k_per, N_OUT)

    def body(x_ref, w_ref, sx_ref, sw_ref, out_ref,
             comm_ref, first_ref,
             send_sems, rs_recv_sems, ag_recv_sems, ready_sems):
        my = lax.axis_index("i")
        left = (my - 1) % N_DEV
        right = (my + 1) % N_DEV

        send_to = (right, left)
        writer = (left, right)

        barrier = pltpu.get_barrier_semaphore()
        for nbr in (left, right):
            pl.semaphore_signal(
                barrier, inc=1,
                device_id=(nbr,), device_id_type=pl.DeviceIdType.MESH,
            )
        pl.semaphore_wait(barrier, 2)

        scale = sx_ref[0] * sw_ref[0]

        def partial_half(c, d):
            xa = x_ref[pl.ds(c * CH, CH), :]
            wa = w_ref[:, d * HALF:(d + 1) * HALF]
            return lax.dot_general(
                xa, wa,
                dimension_numbers=(((1,), (0,)), ((), ())),
                preferred_element_type=jnp.int32,
            )

        def c_recv(s, d):
            return (my - 1 - s) % N_DEV if d == 0 else (my + 1 + s) % N_DEV

        for d in (0, 1):
            first_ref[d] = partial_half(my, d)

        for s in range(N_DEV - 1):
            if s >= SLOTS:
                for d in (0, 1):
                    pl.semaphore_wait(ready_sems.at[d], 1)
            rdmas = []
            for d in (0, 1):
                src = (first_ref.at[d] if s == 0
                       else comm_ref.at[d, (s - 1) % SLOTS])
                rdma = pltpu.make_async_remote_copy(
                    src_ref=src,
                    dst_ref=comm_ref.at[d, s % SLOTS],
                    send_sem=send_sems.at[d, s % 2],
                    recv_sem=rs_recv_sems.at[d, s],
                    device_id=(send_to[d],),
                    device_id_type=pl.DeviceIdType.MESH,
                )
                rdma.start()
                rdmas.append(rdma)
            for d in (0, 1):
                rdmas[d].wait()
            for d in (0, 1):
                comm_ref[d, s % SLOTS] = (
                    comm_ref[d, s % SLOTS] + partial_half(c_recv(s, d), d)
                )
            if 1 <= s <= N_DEV - 5:
                for d in (0, 1):
                    pl.semaphore_signal(
                        ready_sems.at[d], inc=1,
                        device_id=(writer[d],),
                        device_id_type=pl.DeviceIdType.MESH,
                    )

        last_slot = (N_DEV - 2) % SLOTS
        for d in (0, 1):
            g = (my + 1) % N_DEV if d == 0 else (my - 1) % N_DEV
            acc = comm_ref[d, last_slot]
            y = jnp.maximum(acc.astype(jnp.float32) * scale, 0.0)
            out_ref[pl.ds(g * CH, CH), d * HALF:(d + 1) * HALF] = y

        for s in range(N_DEV - 1):
            rdmas = []
            for d in (0, 1):
                c_send = (my + 1 - s) % N_DEV if d == 0 else (my - 1 + s) % N_DEV
                rows = pl.ds(c_send * CH, CH)
                cols = slice(d * HALF, (d + 1) * HALF)
                rdma = pltpu.make_async_remote_copy(
                    src_ref=out_ref.at[rows, cols],
                    dst_ref=out_ref.at[rows, cols],
                    send_sem=send_sems.at[d, s % 2],
                    recv_sem=ag_recv_sems.at[d, s],
                    device_id=(send_to[d],),
                    device_id_type=pl.DeviceIdType.MESH,
                )
                rdma.start()
                rdmas.append(rdma)
            for d in (0, 1):
                rdmas[d].wait()

    return pl.pallas_call(
        body,
        out_shape=jax.ShapeDtypeStruct((M, N_OUT), jnp.float32),
        in_specs=[
            pl.BlockSpec(memory_space=pltpu.VMEM),
            pl.BlockSpec(memory_space=pltpu.VMEM),
            pl.BlockSpec(memory_space=pltpu.SMEM),
            pl.BlockSpec(memory_space=pltpu.SMEM),
        ],
        out_specs=pl.BlockSpec(memory_space=pltpu.VMEM),
        scratch_shapes=[
            pltpu.VMEM((2, SLOTS, CH, HALF), jnp.int32),
            pltpu.VMEM((2, CH, HALF), jnp.int32),
            pltpu.SemaphoreType.DMA((2, 2)),
            pltpu.SemaphoreType.DMA((2, N_DEV - 1)),
            pltpu.SemaphoreType.DMA((2, N_DEV - 1)),
            pltpu.SemaphoreType.REGULAR((2,)),
        ],
        compiler_params=pltpu.CompilerParams(
            collective_id=0,
            vmem_limit_bytes=56 * 1024 * 1024,
        ),
    )(x, w_mat, scale_x, scale_w)


# device time: 844072 ns/iter; 1.0202x vs baseline; 1.0007x over previous
import jax
import jax.numpy as jnp
from jax import lax
from jax.experimental import pallas as pl
from jax.experimental.pallas import tpu as pltpu

N_DEV = 32
M = 4096
N_OUT = 2048
HALF = N_OUT // 2
CH = M // N_DEV
SLOTS = 4


def kernel(x, w_mat, scale_x, scale_w):
    k_per = x.shape[1]
    assert x.shape == (M, k_per)
    assert w_mat.shape == (k_per, N_OUT)

    def body(x_ref, w_ref, sx_ref, sw_ref, out_ref,
             comm_ref, first_ref,
             send_sems, rs_recv_sems, ag_recv_sems, ready_sems):
        my = lax.axis_index("i")
        left = (my - 1) % N_DEV
        right = (my + 1) % N_DEV

        send_to = (right, left)
        writer = (left, right)

        barrier = pltpu.get_barrier_semaphore()
        for nbr in (left, right):
            pl.semaphore_signal(
                barrier, inc=1,
                device_id=(nbr,), device_id_type=pl.DeviceIdType.MESH,
            )
        pl.semaphore_wait(barrier, 2)

        scale = sx_ref[0] * sw_ref[0]

        def partial_half(c, d):
            xa = x_ref[pl.ds(c * CH, CH), :].astype(jnp.bfloat16)
            wa = w_ref[:, d * HALF:(d + 1) * HALF].astype(jnp.bfloat16)
            return lax.dot_general(
                xa, wa,
                dimension_numbers=(((1,), (0,)), ((), ())),
                preferred_element_type=jnp.float32,
            )

        def c_recv(s, d):
            return (my - 1 - s) % N_DEV if d == 0 else (my + 1 + s) % N_DEV

        for d in (0, 1):
            first_ref[d] = partial_half(my, d)

        for s in range(N_DEV - 1):
            if s >= SLOTS:
                for d in (0, 1):
                    pl.semaphore_wait(ready_sems.at[d], 1)
            rdmas = []
            for d in (0, 1):
                src = (first_ref.at[d] if s == 0
                       else comm_ref.at[d, (s - 1) % SLOTS])
                rdma = pltpu.make_async_remote_copy(
                    src_ref=src,
                    dst_ref=comm_ref.at[d, s % SLOTS],
                    send_sem=send_sems.at[d, s % 2],
                    recv_sem=rs_recv_sems.at[d, s],
                    device_id=(send_to[d],),
                    device_id_type=pl.DeviceIdType.MESH,
                )
                rdma.start()
                rdmas.append(rdma)
            for d in (0, 1):
                rdmas[d].wait()
            for d in (0, 1):
                comm_ref[d, s % SLOTS] = (
                    comm_ref[d, s % SLOTS] + partial_half(c_recv(s, d), d)
                )
            if 1 <= s <= N_DEV - 5:
                for d in (0, 1):
                    pl.semaphore_signal(
                        ready_sems.at[d], inc=1,
                        device_id=(writer[d],),
                        device_id_type=pl.DeviceIdType.MESH,
                    )

        last_slot = (N_DEV - 2) % SLOTS
        for d in (0, 1):
            g = (my + 1) % N_DEV if d == 0 else (my - 1) % N_DEV
            acc = comm_ref[d, last_slot]
            y = jnp.maximum(acc * scale, 0.0)
            out_ref[pl.ds(g * CH, CH), d * HALF:(d + 1) * HALF] = y

        for s in range(N_DEV - 1):
            rdmas = []
            for d in (0, 1):
                c_send = (my + 1 - s) % N_DEV if d == 0 else (my - 1 + s) % N_DEV
                rows = pl.ds(c_send * CH, CH)
                cols = slice(d * HALF, (d + 1) * HALF)
                rdma = pltpu.make_async_remote_copy(
                    src_ref=out_ref.at[rows, cols],
                    dst_ref=out_ref.at[rows, cols],
                    send_sem=send_sems.at[d, s % 2],
                    recv_sem=ag_recv_sems.at[d, s],
                    device_id=(send_to[d],),
                    device_id_type=pl.DeviceIdType.MESH,
                )
                rdma.start()
                rdmas.append(rdma)
            for d in (0, 1):
                rdmas[d].wait()

    return pl.pallas_call(
        body,
        out_shape=jax.ShapeDtypeStruct((M, N_OUT), jnp.float32),
        in_specs=[
            pl.BlockSpec(memory_space=pltpu.VMEM),
            pl.BlockSpec(memory_space=pltpu.VMEM),
            pl.BlockSpec(memory_space=pltpu.SMEM),
            pl.BlockSpec(memory_space=pltpu.SMEM),
        ],
        out_specs=pl.BlockSpec(memory_space=pltpu.VMEM),
        scratch_shapes=[
            pltpu.VMEM((2, SLOTS, CH, HALF), jnp.float32),
            pltpu.VMEM((2, CH, HALF), jnp.float32),
            pltpu.SemaphoreType.DMA((2, 2)),
            pltpu.SemaphoreType.DMA((2, N_DEV - 1)),
            pltpu.SemaphoreType.DMA((2, N_DEV - 1)),
            pltpu.SemaphoreType.REGULAR((2,)),
        ],
        compiler_params=pltpu.CompilerParams(
            collective_id=0,
            vmem_limit_bytes=56 * 1024 * 1024,
        ),
    )(x, w_mat, scale_x, scale_w)


# device time: 505511 ns/iter; 1.7034x vs baseline; 1.6697x over previous
import jax
import jax.numpy as jnp
from jax import lax
from jax.experimental import pallas as pl
from jax.experimental.pallas import tpu as pltpu

N_DEV = 32
M = 4096
N_OUT = 2048
HALF = N_OUT // 2
CH = M // N_DEV
SLOTS = 4

_W_ORDER = [(0, 0), (1, 0), (1, 1), (0, 1), (0, 2), (1, 2), (1, 3), (0, 3)]
_LOGICAL = {(x, y, z): z * 8 + w for z in range(4)
            for w, (x, y) in enumerate(_W_ORDER)}
_C16 = [(0, 0), (0, 1), (0, 2), (0, 3), (1, 3), (1, 2), (1, 1), (2, 1),
        (2, 2), (2, 3), (3, 3), (3, 2), (3, 1), (3, 0), (2, 0), (1, 0)]
_RING_COORDS = ([(0, y, z) for (y, z) in _C16]
                + [(1, y, z) for (y, z) in reversed(_C16)])
RING = [_LOGICAL[c] for c in _RING_COORDS]
POS = [RING.index(l) for l in range(N_DEV)]


def kernel(x, w_mat, scale_x, scale_w):
    k_per = x.shape[1]
    assert x.shape == (M, k_per)
    assert w_mat.shape == (k_per, N_OUT)

    def body(x_ref, w_ref, sx_ref, sw_ref, ring_ref, pos_ref, out_ref,
             comm_ref, first_ref,
             send_sems, rs_recv_sems, ag_recv_sems, ready_sems):
        my = lax.axis_index("i")

        pos = pos_ref[my]
        right = ring_ref[(pos + 1) % N_DEV]
        left = ring_ref[(pos - 1) % N_DEV]

        send_to = (right, left)
        writer = (left, right)

        barrier = pltpu.get_barrier_semaphore()
        for nbr in (left, right):
            pl.semaphore_signal(
                barrier, inc=1,
                device_id=(nbr,), device_id_type=pl.DeviceIdType.MESH,
            )
        pl.semaphore_wait(barrier, 2)

        scale = sx_ref[0] * sw_ref[0]

        def partial_half(c, d):
            xa = x_ref[pl.ds(c * CH, CH), :]
            wa = w_ref[:, d * HALF:(d + 1) * HALF]
            return lax.dot_general(
                xa, wa,
                dimension_numbers=(((1,), (0,)), ((), ())),
                preferred_element_type=jnp.int32,
            )

        def c_recv(s, d):
            return (pos - 1 - s) % N_DEV if d == 0 else (pos + 1 + s) % N_DEV

        for d in (0, 1):
            first_ref[d] = partial_half(pos, d)

        for s in range(N_DEV - 1):
            if s >= SLOTS:
                for d in (0, 1):
                    pl.semaphore_wait(ready_sems.at[d], 1)
            rdmas = []
            for d in (0, 1):
                src = (first_ref.at[d] if s == 0
                       else comm_ref.at[d, (s - 1) % SLOTS])
                rdma = pltpu.make_async_remote_copy(
                    src_ref=src,
                    dst_ref=comm_ref.at[d, s % SLOTS],
                    send_sem=send_sems.at[d, s % 2],
                    recv_sem=rs_recv_sems.at[d, s],
                    device_id=(send_to[d],),
                    device_id_type=pl.DeviceIdType.MESH,
                )
                rdma.start()
                rdmas.append(rdma)
            for d in (0, 1):
                rdmas[d].wait()
            for d in (0, 1):
                comm_ref[d, s % SLOTS] = (
                    comm_ref[d, s % SLOTS] + partial_half(c_recv(s, d), d)
                )
            if 1 <= s <= N_DEV - 5:
                for d in (0, 1):
                    pl.semaphore_signal(
                        ready_sems.at[d], inc=1,
                        device_id=(writer[d],),
                        device_id_type=pl.DeviceIdType.MESH,
                    )

        last_slot = (N_DEV - 2) % SLOTS
        for d in (0, 1):
            g = (pos + 1) % N_DEV if d == 0 else (pos - 1) % N_DEV
            acc = comm_ref[d, last_slot]
            y = jnp.maximum(acc.astype(jnp.float32) * scale, 0.0)
            out_ref[pl.ds(g * CH, CH), d * HALF:(d + 1) * HALF] = y

        for s in range(N_DEV - 1):
            rdmas = []
            for d in (0, 1):
                c_send = (pos + 1 - s) % N_DEV if d == 0 else (pos - 1 + s) % N_DEV
                rows = pl.ds(c_send * CH, CH)
                cols = slice(d * HALF, (d + 1) * HALF)
                rdma = pltpu.make_async_remote_copy(
                    src_ref=out_ref.at[rows, cols],
                    dst_ref=out_ref.at[rows, cols],
                    send_sem=send_sems.at[d, s % 2],
                    recv_sem=ag_recv_sems.at[d, s],
                    device_id=(send_to[d],),
                    device_id_type=pl.DeviceIdType.MESH,
                )
                rdma.start()
                rdmas.append(rdma)
            for d in (0, 1):
                rdmas[d].wait()

    return pl.pallas_call(
        body,
        out_shape=jax.ShapeDtypeStruct((M, N_OUT), jnp.float32),
        in_specs=[
            pl.BlockSpec(memory_space=pltpu.VMEM),
            pl.BlockSpec(memory_space=pltpu.VMEM),
            pl.BlockSpec(memory_space=pltpu.SMEM),
            pl.BlockSpec(memory_space=pltpu.SMEM),
            pl.BlockSpec(memory_space=pltpu.SMEM),
            pl.BlockSpec(memory_space=pltpu.SMEM),
        ],
        out_specs=pl.BlockSpec(memory_space=pltpu.VMEM),
        scratch_shapes=[
            pltpu.VMEM((2, SLOTS, CH, HALF), jnp.int32),
            pltpu.VMEM((2, CH, HALF), jnp.int32),
            pltpu.SemaphoreType.DMA((2, 2)),
            pltpu.SemaphoreType.DMA((2, N_DEV - 1)),
            pltpu.SemaphoreType.DMA((2, N_DEV - 1)),
            pltpu.SemaphoreType.REGULAR((2,)),
        ],
        compiler_params=pltpu.CompilerParams(
            collective_id=0,
            vmem_limit_bytes=56 * 1024 * 1024,
        ),
    )(x, w_mat, scale_x, scale_w,
      jnp.array(RING, dtype=jnp.int32), jnp.array(POS, dtype=jnp.int32))
